# Initial kernel scaffold; baseline (speedup 1.0000x reference)
import jax
import jax.numpy as jnp
from jax.experimental import pallas as pl


def kernel(input_ids, base_weight, bias):
    raise NotImplementedError("write your pallas kernel here")


# speedup vs baseline: 5.4373x; 5.4373x over previous
import functools

import jax
import jax.numpy as jnp
from jax import lax
from jax.experimental import pallas as pl
from jax.experimental.pallas import tpu as pltpu
from jax.experimental.pallas import tpu_sc as plsc

DIM = 64
LANES = 16
NUM_WORKERS = 32
CHUNK = 128


def _sc_embed(ids_flat, base_weight, bias):
    n = ids_flat.shape[0]
    per_w = n // NUM_WORKERS
    n_chunks = per_w // CHUNK
    mesh = plsc.VectorSubcoreMesh(core_axis_name="c", subcore_axis_name="s")

    @functools.partial(
        pl.kernel,
        mesh=mesh,
        out_type=jax.ShapeDtypeStruct((n, DIM), jnp.float32),
        scratch_types=[
            pltpu.VMEM((CHUNK,), jnp.int32),
            pltpu.VMEM((CHUNK, DIM), jnp.float32),
            pltpu.VMEM((CHUNK, DIM), jnp.float32),
            pltpu.SemaphoreType.DMA,
            pltpu.SemaphoreType.DMA,
        ],
        compiler_params=pltpu.CompilerParams(use_tc_tiling_on_sc=False),
    )
    def k(ids_hbm, base_hbm, bias_hbm, out_hbm, idx_v, acc_v, tmp_v, sem0, sem1):
        wid = lax.axis_index("s") * 2 + lax.axis_index("c")
        base_off = wid * per_w

        def chunk_body(c, carry):
            off = base_off + c * CHUNK
            pltpu.sync_copy(ids_hbm.at[pl.ds(off, CHUNK)], idx_v)
            cp0 = pltpu.async_copy(base_hbm.at[idx_v], acc_v, sem0)
            cp1 = pltpu.async_copy(bias_hbm.at[idx_v], tmp_v, sem1)
            cp0.wait()
            cp1.wait()

            def add_row(r, carry2):
                for j in range(DIM // LANES):
                    sl = pl.ds(j * LANES, LANES)
                    acc_v[r, sl] = acc_v[r, sl] + tmp_v[r, sl]
                return carry2

            lax.fori_loop(0, CHUNK, add_row, 0)
            pltpu.sync_copy(acc_v, out_hbm.at[pl.ds(off, CHUNK)])
            return carry

        lax.fori_loop(0, n_chunks, chunk_body, 0)

    return k(ids_flat, base_weight, bias)


def kernel(input_ids, base_weight, bias):
    b, l = input_ids.shape
    ids_flat = input_ids.reshape(-1).astype(jnp.int32)
    out = _sc_embed(ids_flat, base_weight, bias)
    return out.reshape(b, l, DIM)
